# baseline (device time: 32380 ns/iter reference)
import jax
import jax.numpy as jnp
from jax import lax
from jax.experimental import pallas as pl
from jax.experimental.pallas import tpu as pltpu

N_DEV = 4
B_PER = 128
D = 128
H_PER = 256
N_PHASES = 6


def kernel(x, Win0, Wout0, Win1, Wout1, Win2, Wout2):
    def body(x_ref, win0_ref, wout0_ref, win1_ref, wout1_ref, win2_ref,
             wout2_ref, out_ref, xfull_ref, pbuf_ref, acc_ref,
             send_sems, recv_sems):
        my = lax.axis_index("i")
        my_row = my * B_PER

        barrier_sem = pltpu.get_barrier_semaphore()
        for off in range(1, N_DEV):
            peer = lax.rem(my + off, N_DEV)
            pl.semaphore_signal(
                barrier_sem, inc=1,
                device_id=(peer,), device_id_type=pl.DeviceIdType.MESH,
            )
        pl.semaphore_wait(barrier_sem, N_DEV - 1)

        def exchange(p, src_for, dst_for):
            rdmas = []
            for off in range(1, N_DEV):
                peer = lax.rem(my + off, N_DEV)
                rdma = pltpu.make_async_remote_copy(
                    src_ref=src_for(off, peer),
                    dst_ref=dst_for(off, peer),
                    send_sem=send_sems.at[p, off - 1],
                    recv_sem=recv_sems.at[p, off - 1],
                    device_id=(peer,),
                    device_id_type=pl.DeviceIdType.MESH,
                )
                rdma.start()
                rdmas.append(rdma)
            for rdma in rdmas:
                rdma.wait()

        xfull_ref[pl.ds(my_row, B_PER), :] = x_ref[:, :]
        exchange(
            0,
            lambda off, peer: x_ref,
            lambda off, peer: xfull_ref.at[pl.ds(my_row, B_PER), :],
        )

        wins = [win0_ref, win1_ref, win2_ref]
        wouts = [wout0_ref, wout1_ref, wout2_ref]

        for k in range(3):
            h = jnp.maximum(
                jnp.dot(xfull_ref[:, :], wins[k][:, :],
                        preferred_element_type=jnp.float32),
                0.0,
            )
            pbuf_ref[:, :] = jnp.dot(h, wouts[k][:, :],
                                     preferred_element_type=jnp.float32)

            exchange(
                2 * k + 1,
                lambda off, peer: pbuf_ref.at[pl.ds(peer * B_PER, B_PER), :],
                lambda off, peer: acc_ref.at[off - 1],
            )
            x_next = (
                pbuf_ref[pl.ds(my_row, B_PER), :]
                + acc_ref[0] + acc_ref[1] + acc_ref[2]
            )

            if k == 2:
                out_ref[:, :] = x_next
            else:
                xfull_ref[pl.ds(my_row, B_PER), :] = x_next
                exchange(
                    2 * k + 2,
                    lambda off, peer: xfull_ref.at[pl.ds(my_row, B_PER), :],
                    lambda off, peer: xfull_ref.at[pl.ds(my_row, B_PER), :],
                )

    return pl.pallas_call(
        body,
        out_shape=jax.ShapeDtypeStruct((B_PER, D), jnp.float32),
        in_specs=[pl.BlockSpec(memory_space=pltpu.VMEM)] * 7,
        out_specs=pl.BlockSpec(memory_space=pltpu.VMEM),
        scratch_shapes=[
            pltpu.VMEM((N_DEV * B_PER, D), jnp.float32),
            pltpu.VMEM((N_DEV * B_PER, D), jnp.float32),
            pltpu.VMEM((N_DEV - 1, B_PER, D), jnp.float32),
            pltpu.SemaphoreType.DMA((N_PHASES, N_DEV - 1)),
            pltpu.SemaphoreType.DMA((N_PHASES, N_DEV - 1)),
        ],
        compiler_params=pltpu.CompilerParams(collective_id=0),
    )(x, Win0, Wout0, Win1, Wout1, Win2, Wout2)


# device time: 31473 ns/iter; 1.0288x vs baseline; 1.0288x over previous
import jax
import jax.numpy as jnp
from jax import lax
from jax.experimental import pallas as pl
from jax.experimental.pallas import tpu as pltpu

N_DEV = 4
B_PER = 128
D = 128
H_PER = 256
N_PHASES = 6


def kernel(x, Win0, Wout0, Win1, Wout1, Win2, Wout2):
    def body(x_ref, win0_ref, wout0_ref, win1_ref, wout1_ref, win2_ref,
             wout2_ref, out_ref, xfull_ref, pbuf_ref, acc_ref,
             send_sems, recv_sems):
        my = lax.axis_index("i")
        my_row = my * B_PER

        barrier_sem = pltpu.get_barrier_semaphore()
        for off in range(1, N_DEV):
            peer = lax.rem(my + off, N_DEV)
            pl.semaphore_signal(
                barrier_sem, inc=1,
                device_id=(peer,), device_id_type=pl.DeviceIdType.MESH,
            )
        pl.semaphore_wait(barrier_sem, N_DEV - 1)

        def make_rdma(src_ref, dst_ref, p, slot, peer):
            return pltpu.make_async_remote_copy(
                src_ref=src_ref,
                dst_ref=dst_ref,
                send_sem=send_sems.at[p, slot],
                recv_sem=recv_sems.at[p, slot],
                device_id=(peer,),
                device_id_type=pl.DeviceIdType.MESH,
            )

        wins = [win0_ref, win1_ref, win2_ref]
        wouts = [wout0_ref, wout1_ref, wout2_ref]

        xfull_ref[pl.ds(my_row, B_PER), :] = x_ref[:, :]
        pending_ag = []
        for off in range(1, N_DEV):
            peer = lax.rem(my + off, N_DEV)
            d = make_rdma(x_ref, xfull_ref.at[pl.ds(my_row, B_PER), :],
                          0, off - 1, peer)
            d.start()
            pending_ag.append(d)

        x_next = x_ref[:, :]
        pending_rs = {}
        for k in range(3):
            ag_p, rs_p = 2 * k, 2 * k + 1

            h_me = jnp.maximum(
                jnp.dot(x_next, wins[k][:, :],
                        preferred_element_type=jnp.float32), 0.0)
            p_me = jnp.dot(h_me, wouts[k][:, :],
                           preferred_element_type=jnp.float32)

            for off in range(1, N_DEV):
                s = lax.rem(my - off + N_DEV, N_DEV)
                s_row = s * B_PER
                recv_d = make_rdma(
                    xfull_ref.at[pl.ds(s_row, B_PER), :],
                    xfull_ref.at[pl.ds(s_row, B_PER), :],
                    ag_p, off - 1, s)
                recv_d.wait_recv()

                h_s = jnp.maximum(
                    jnp.dot(xfull_ref[pl.ds(s_row, B_PER), :], wins[k][:, :],
                            preferred_element_type=jnp.float32), 0.0)
                if off in pending_rs:
                    pending_rs[off].wait_send()
                pbuf_ref[pl.ds(s_row, B_PER), :] = jnp.dot(
                    h_s, wouts[k][:, :], preferred_element_type=jnp.float32)

                d = make_rdma(pbuf_ref.at[pl.ds(s_row, B_PER), :],
                              acc_ref.at[3 - off], rs_p, 3 - off, s)
                d.start()
                pending_rs[off] = d

            for off in range(1, N_DEV):
                a = lax.rem(my - off + N_DEV, N_DEV)
                recv_d = make_rdma(acc_ref.at[off - 1], acc_ref.at[off - 1],
                                   rs_p, off - 1, a)
                recv_d.wait_recv()
            x_next = p_me + acc_ref[0] + acc_ref[1] + acc_ref[2]

            if k < 2:
                for d in pending_ag:
                    d.wait_send()
                xfull_ref[pl.ds(my_row, B_PER), :] = x_next
                pending_ag = []
                for off in range(1, N_DEV):
                    peer = lax.rem(my + off, N_DEV)
                    d = make_rdma(xfull_ref.at[pl.ds(my_row, B_PER), :],
                                  xfull_ref.at[pl.ds(my_row, B_PER), :],
                                  2 * k + 2, off - 1, peer)
                    d.start()
                    pending_ag.append(d)

        out_ref[:, :] = x_next

        for d in pending_ag:
            d.wait_send()
        for d in pending_rs.values():
            d.wait_send()

    return pl.pallas_call(
        body,
        out_shape=jax.ShapeDtypeStruct((B_PER, D), jnp.float32),
        in_specs=[pl.BlockSpec(memory_space=pltpu.VMEM)] * 7,
        out_specs=pl.BlockSpec(memory_space=pltpu.VMEM),
        scratch_shapes=[
            pltpu.VMEM((N_DEV * B_PER, D), jnp.float32),
            pltpu.VMEM((N_DEV * B_PER, D), jnp.float32),
            pltpu.VMEM((N_DEV - 1, B_PER, D), jnp.float32),
            pltpu.SemaphoreType.DMA((N_PHASES, N_DEV - 1)),
            pltpu.SemaphoreType.DMA((N_PHASES, N_DEV - 1)),
        ],
        compiler_params=pltpu.CompilerParams(collective_id=0),
    )(x, Win0, Wout0, Win1, Wout1, Win2, Wout2)


# device time: 20856 ns/iter; 1.5526x vs baseline; 1.5091x over previous
import jax
import jax.numpy as jnp
from jax import lax
from jax.experimental import pallas as pl
from jax.experimental.pallas import tpu as pltpu

N_DEV = 4
B_PER = 128
D = 128
H_PER = 256


def kernel(x, Win0, Wout0, Win1, Wout1, Win2, Wout2):
    def body(x_ref, win0_ref, wout0_ref, win1_ref, wout1_ref, win2_ref,
             wout2_ref, out_ref, wpair_send, pair_bf, send_sems, recv_sems):
        my = lax.axis_index("i")

        wins = [win0_ref, win1_ref, win2_ref]
        wouts = [wout0_ref, wout1_ref, wout2_ref]

        for k in range(3):
            wpair_send[k, 0:H_PER, :] = wins[k][:, :].astype(jnp.bfloat16).T
            wpair_send[k, H_PER:2 * H_PER, :] = wouts[k][:, :].astype(
                jnp.bfloat16)

        barrier_sem = pltpu.get_barrier_semaphore()
        for off in range(1, N_DEV):
            peer = lax.rem(my + off, N_DEV)
            pl.semaphore_signal(
                barrier_sem, inc=1,
                device_id=(peer,), device_id_type=pl.DeviceIdType.MESH,
            )
        pl.semaphore_wait(barrier_sem, N_DEV - 1)

        def make_rdma(src_ref, dst_ref, p, slot, peer):
            return pltpu.make_async_remote_copy(
                src_ref=src_ref,
                dst_ref=dst_ref,
                send_sem=send_sems.at[p, slot],
                recv_sem=recv_sems.at[p, slot],
                device_id=(peer,),
                device_id_type=pl.DeviceIdType.MESH,
            )

        sends = []
        for k in range(3):
            for off in range(1, N_DEV):
                peer = lax.rem(my + off, N_DEV)
                d = make_rdma(wpair_send.at[k], pair_bf.at[k, off - 1],
                              k, off - 1, peer)
                d.start()
                sends.append(d)

        x_cur = x_ref[:, :]
        for k in range(3):
            h = jnp.maximum(
                jnp.dot(x_cur, wins[k][:, :],
                        preferred_element_type=jnp.float32), 0.0)
            acc = jnp.dot(h, wouts[k][:, :],
                          preferred_element_type=jnp.float32)
            x16 = x_cur.astype(jnp.bfloat16)
            for off in range(1, N_DEV):
                slot = off - 1
                make_rdma(pair_bf.at[k, slot], pair_bf.at[k, slot],
                          k, slot, my).wait_recv()
                wT = pair_bf[k, slot, 0:H_PER, :]
                wo = pair_bf[k, slot, H_PER:2 * H_PER, :]
                h_s = jnp.maximum(
                    lax.dot_general(
                        x16, wT, (((1,), (1,)), ((), ())),
                        preferred_element_type=jnp.float32), 0.0)
                acc = acc + jnp.dot(h_s.astype(jnp.bfloat16), wo,
                                    preferred_element_type=jnp.float32)
            x_cur = acc

        out_ref[:, :] = x_cur

        for d in sends:
            d.wait_send()

    return pl.pallas_call(
        body,
        out_shape=jax.ShapeDtypeStruct((B_PER, D), jnp.float32),
        in_specs=[pl.BlockSpec(memory_space=pltpu.VMEM)] * 7,
        out_specs=pl.BlockSpec(memory_space=pltpu.VMEM),
        scratch_shapes=[
            pltpu.VMEM((3, 2 * H_PER, D), jnp.bfloat16),
            pltpu.VMEM((3, N_DEV - 1, 2 * H_PER, D), jnp.bfloat16),
            pltpu.SemaphoreType.DMA((3, N_DEV - 1)),
            pltpu.SemaphoreType.DMA((3, N_DEV - 1)),
        ],
        compiler_params=pltpu.CompilerParams(collective_id=0),
    )(x, Win0, Wout0, Win1, Wout1, Win2, Wout2)
